# baseline (speedup 1.0000x reference)
import functools

import jax
import jax.numpy as jnp
from jax import lax
from jax.experimental import pallas as pl
from jax.experimental.pallas import tpu as pltpu
from jax.experimental.pallas import tpu_sc as plsc

B, S, D = 4, 8192, 1024
NC, NS = 2, 16
NW = NC * NS
SW = S // NW
R = 4
NCHUNK = SW // R
NSLOT = 6
LOOK = 3
NTAIL = NCHUNK % NSLOT


def _body(in_hbm, pos_hbm, out_hbm, *scratch):
    io_bufs = scratch[0:NSLOT]
    pos_bufs = scratch[NSLOT:2 * NSLOT]
    in_sems = scratch[2 * NSLOT:3 * NSLOT]
    out_sems = scratch[3 * NSLOT:4 * NSLOT]

    cid = lax.axis_index("c")
    sid = lax.axis_index("s")
    wid = sid * NC + cid
    s_base = wid * SW

    def issue_loads(u, slot):
        s0 = s_base + u * R
        pltpu.async_copy(pos_hbm.at[pl.ds(s0, R)], pos_bufs[slot], in_sems[slot])
        pltpu.async_copy(
            in_hbm.at[:, pl.ds(s0, R)], io_bufs[slot], in_sems[slot]
        )

    def wait_loads(u, slot):
        s0 = s_base + u * R
        pltpu.make_async_copy(
            pos_hbm.at[pl.ds(s0, R)], pos_bufs[slot], in_sems[slot]
        ).wait()
        pltpu.make_async_copy(
            in_hbm.at[:, pl.ds(s0, R)], io_bufs[slot], in_sems[slot]
        ).wait()

    def issue_outs(u, slot):
        s0 = s_base + u * R
        pltpu.async_copy(
            io_bufs[slot], out_hbm.at[:, pl.ds(s0, R)], out_sems[slot]
        )

    def wait_outs(u, slot):
        s0 = s_base + u * R
        pltpu.make_async_copy(
            io_bufs[slot], out_hbm.at[:, pl.ds(s0, R)], out_sems[slot]
        ).wait()

    def compute(slot):
        io = io_bufs[slot]
        pos = pos_bufs[slot]
        for r in range(R):
            @plsc.parallel_loop(0, D // 16, unroll=8)
            def _(i):
                o = i * 16
                sl = pl.ds(o, 16)
                p = pos[r, sl]
                for b in range(B):
                    plsc.addupdate(io.at[b, r, sl], p)

    def step(u, slot):
        @pl.when(u >= NSLOT - LOOK)
        def _():
            wait_outs(u - (NSLOT - LOOK), (slot + LOOK) % NSLOT)

        @pl.when(u < NCHUNK - LOOK)
        def _():
            issue_loads(u + LOOK, (slot + LOOK) % NSLOT)

        wait_loads(u, slot)
        compute(slot)
        issue_outs(u, slot)

    for k in range(LOOK):
        issue_loads(jnp.int32(k), k)

    def group(i, _):
        for slot in range(NSLOT):
            step(i * NSLOT + slot, slot)
        return 0

    lax.fori_loop(0, NCHUNK // NSLOT, group, 0)

    for k in range(NTAIL):
        u = NCHUNK - NTAIL + k
        step(jnp.int32(u), u % NSLOT)

    for k in range(NSLOT - LOOK):
        u = NCHUNK - (NSLOT - LOOK) + k
        wait_outs(u, u % NSLOT)


@jax.jit
def _sc_add(inp, pos):
    mesh = plsc.VectorSubcoreMesh(core_axis_name="c", subcore_axis_name="s")
    return pl.kernel(
        _body,
        mesh=mesh,
        out_type=jax.ShapeDtypeStruct((B, S, D), jnp.float32),
        scratch_types=(
            [pltpu.VMEM((B, R, D), jnp.float32) for _ in range(NSLOT)]
            + [pltpu.VMEM((R, D), jnp.float32) for _ in range(NSLOT)]
            + [pltpu.SemaphoreType.DMA for _ in range(2 * NSLOT)]
        ),
    )(inp, pos)


def kernel(input_embeddings, pos_table):
    return _sc_add(input_embeddings, pos_table)


# speedup vs baseline: 1.0047x; 1.0047x over previous
import functools

import jax
import jax.numpy as jnp
from jax import lax
from jax.experimental import pallas as pl
from jax.experimental.pallas import tpu as pltpu
from jax.experimental.pallas import tpu_sc as plsc

B, S, D = 4, 8192, 1024
NC, NS = 2, 16
NW = NC * NS
SW = S // NW
R = 4
NCHUNK = SW // R
NSLOT = 6
LOOK = 4
NTAIL = NCHUNK % NSLOT


def _body(in_hbm, pos_hbm, out_hbm, *scratch):
    io_bufs = scratch[0:NSLOT]
    pos_bufs = scratch[NSLOT:2 * NSLOT]
    in_sems = scratch[2 * NSLOT:3 * NSLOT]
    out_sems = scratch[3 * NSLOT:4 * NSLOT]

    cid = lax.axis_index("c")
    sid = lax.axis_index("s")
    wid = sid * NC + cid
    s_base = wid * SW

    def issue_loads(u, slot):
        s0 = s_base + u * R
        pltpu.async_copy(pos_hbm.at[pl.ds(s0, R)], pos_bufs[slot], in_sems[slot])
        pltpu.async_copy(
            in_hbm.at[:, pl.ds(s0, R)], io_bufs[slot], in_sems[slot]
        )

    def wait_loads(u, slot):
        s0 = s_base + u * R
        pltpu.make_async_copy(
            pos_hbm.at[pl.ds(s0, R)], pos_bufs[slot], in_sems[slot]
        ).wait()
        pltpu.make_async_copy(
            in_hbm.at[:, pl.ds(s0, R)], io_bufs[slot], in_sems[slot]
        ).wait()

    def issue_outs(u, slot):
        s0 = s_base + u * R
        pltpu.async_copy(
            io_bufs[slot], out_hbm.at[:, pl.ds(s0, R)], out_sems[slot]
        )

    def wait_outs(u, slot):
        s0 = s_base + u * R
        pltpu.make_async_copy(
            io_bufs[slot], out_hbm.at[:, pl.ds(s0, R)], out_sems[slot]
        ).wait()

    def compute(slot):
        io = io_bufs[slot]
        pos = pos_bufs[slot]
        for r in range(R):
            @plsc.parallel_loop(0, D // 16, unroll=8)
            def _(i):
                o = i * 16
                sl = pl.ds(o, 16)
                p = pos[r, sl]
                for b in range(B):
                    plsc.addupdate(io.at[b, r, sl], p)

    def step(u, slot):
        @pl.when(u >= NSLOT - LOOK)
        def _():
            wait_outs(u - (NSLOT - LOOK), (slot + LOOK) % NSLOT)

        @pl.when(u < NCHUNK - LOOK)
        def _():
            issue_loads(u + LOOK, (slot + LOOK) % NSLOT)

        wait_loads(u, slot)
        compute(slot)
        issue_outs(u, slot)

    for k in range(LOOK):
        issue_loads(jnp.int32(k), k)

    def group(i, _):
        for slot in range(NSLOT):
            step(i * NSLOT + slot, slot)
        return 0

    lax.fori_loop(0, NCHUNK // NSLOT, group, 0)

    for k in range(NTAIL):
        u = NCHUNK - NTAIL + k
        step(jnp.int32(u), u % NSLOT)

    for k in range(NSLOT - LOOK):
        u = NCHUNK - (NSLOT - LOOK) + k
        wait_outs(u, u % NSLOT)


@jax.jit
def _sc_add(inp, pos):
    mesh = plsc.VectorSubcoreMesh(core_axis_name="c", subcore_axis_name="s")
    return pl.kernel(
        _body,
        mesh=mesh,
        out_type=jax.ShapeDtypeStruct((B, S, D), jnp.float32),
        scratch_types=(
            [pltpu.VMEM((B, R, D), jnp.float32) for _ in range(NSLOT)]
            + [pltpu.VMEM((R, D), jnp.float32) for _ in range(NSLOT)]
            + [pltpu.SemaphoreType.DMA for _ in range(2 * NSLOT)]
        ),
    )(inp, pos)


def kernel(input_embeddings, pos_table):
    return _sc_add(input_embeddings, pos_table)
